# baseline (speedup 1.0000x reference)
import jax
import jax.numpy as jnp
from jax.experimental import pallas as pl
from jax.experimental.pallas import tpu as pltpu

NUM_CHANNELS = 8
EPOCH_ALPHA = 0.3
B = 8192
STYLE_DIM = 4096
GROUP = STYLE_DIM // NUM_CHANNELS


ROWS_PER_STEP = 64
LANES = 128
N_ROW_STEPS = B // ROWS_PER_STEP
N_COL_ITERS = B // LANES

_ROT_A = (13, 15, 26, 6)
_ROT_B = (17, 29, 16, 24)


def _perm_kernel(keys_ref, cand_lab_ref, row_lab_ref, out_ref):
    k = pl.program_id(0)
    s = pl.program_id(1)
    row_base = s * ROWS_PER_STEP

    k1 = keys_ref[k, 0]
    k2 = keys_ref[k, 1]
    k3 = k1 ^ k2 ^ jnp.uint32(0x1BD11BDA)
    ks = (k1, k2, k3)

    row_lab = row_lab_ref[0]
    rows_i32 = (jax.lax.broadcasted_iota(jnp.int32, (ROWS_PER_STEP, 1), 0)
                + row_base)
    row_term = rows_i32.astype(jnp.uint32) * jnp.uint32(B)
    lane_u = jax.lax.broadcasted_iota(jnp.uint32, (1, LANES), 1)
    lane_i32 = jax.lax.broadcasted_iota(jnp.int32, (1, LANES), 1)

    def one_chunk(jj):
        jbase = jj * LANES
        cand_lab = cand_lab_ref[jj, 0]
        x1 = row_term + (lane_u + jnp.uint32(jbase))
        x0 = jnp.full((ROWS_PER_STEP, LANES), ks[0], dtype=jnp.uint32)
        x1 = x1 + ks[1]
        for i in range(5):
            rots = _ROT_A if i % 2 == 0 else _ROT_B
            for r in rots:
                x0 = x0 + x1
                x1 = ((x1 << jnp.uint32(r)) | (x1 >> jnp.uint32(32 - r)))
                x1 = x1 ^ x0
            x0 = x0 + ks[(i + 1) % 3]
            x1 = x1 + (ks[(i + 2) % 3] + jnp.uint32(i + 1))
        score = ((x0 ^ x1) >> jnp.uint32(9)).astype(jnp.int32)
        valid = cand_lab[None, :] != row_lab
        return jnp.where(valid, score, -1)

    UNROLL = 2

    def body(g, carry):
        run_max, run_idx = carry
        for u in range(UNROLL):
            jj = g * UNROLL + u
            masked = one_chunk(jj)
            take = masked > run_max
            run_max = jnp.maximum(run_max, masked)
            run_idx = jnp.where(take, lane_i32 + jj * LANES, run_idx)
        return (run_max, run_idx)

    init = (jnp.full((ROWS_PER_STEP, LANES), -1, jnp.int32),
            jnp.zeros((ROWS_PER_STEP, LANES), jnp.int32))
    run_max, run_idx = jax.lax.fori_loop(0, N_COL_ITERS // UNROLL, body, init)
    rmax = jnp.max(run_max, axis=1, keepdims=True)
    ridx = jnp.min(jnp.where(run_max == rmax, run_idx, jnp.int32(B)),
                   axis=1, keepdims=True)
    out_ref[0, 0] = jnp.where(rmax >= 0, ridx, rows_i32)


def _compute_perm(keys, labels):
    cand_lab = labels.reshape(N_COL_ITERS, 1, LANES)
    row_lab = labels.reshape(N_ROW_STEPS, ROWS_PER_STEP, 1)
    out = pl.pallas_call(
        _perm_kernel,
        grid=(NUM_CHANNELS, N_ROW_STEPS),
        in_specs=[
            pl.BlockSpec((NUM_CHANNELS, 2), lambda k, s: (0, 0),
                         memory_space=pltpu.SMEM),
            pl.BlockSpec((N_COL_ITERS, 1, LANES), lambda k, s: (0, 0, 0)),
            pl.BlockSpec((1, ROWS_PER_STEP, 1), lambda k, s: (s, 0, 0)),
        ],
        out_specs=pl.BlockSpec((1, 1, ROWS_PER_STEP, 1),
                               lambda k, s: (k, s, 0, 0)),
        out_shape=jax.ShapeDtypeStruct(
            (NUM_CHANNELS, N_ROW_STEPS, ROWS_PER_STEP, 1), jnp.int32),
        compiler_params=pltpu.CompilerParams(
            dimension_semantics=("parallel", "arbitrary")),
    )(keys, cand_lab, row_lab)
    return out.reshape(NUM_CHANNELS, B)


ROW_BLOCK = 1024
N_ROW_BLOCKS = B // ROW_BLOCK
GATHER_UNROLL = 8


ROWS_PER_MIX = ROW_BLOCK // ROWS_PER_STEP


def _mix_kernel(src_ref, perm_ref, lam_ref, out_ref, donor_ref):
    rb = pl.program_id(1)
    row_start = rb * ROW_BLOCK

    def body(g, _):
        for u in range(GATHER_UNROLL):
            mi = g * GATHER_UNROLL + u
            didx = perm_ref[0, 0, 0, mi]
            donor_ref[mi, 0] = src_ref[didx, 0, 0]
        return 0

    jax.lax.fori_loop(0, ROW_BLOCK // GATHER_UNROLL, body, 0)

    lam = lam_ref[0, 0]
    self_blk = src_ref[pl.ds(row_start, ROW_BLOCK), 0, 0, :]
    donor_blk = donor_ref[:, 0, :]
    out_ref[:, 0, 0, :] = lam * self_blk + (1.0 - lam) * donor_blk


def _mix(z4, perm4, lam4):
    return pl.pallas_call(
        _mix_kernel,
        grid=(NUM_CHANNELS, N_ROW_BLOCKS),
        in_specs=[
            pl.BlockSpec((B, 1, 1, GROUP), lambda k, rb: (0, k, 0, 0)),
            pl.BlockSpec((1, 1, 1, ROW_BLOCK), lambda k, rb: (k, rb, 0, 0),
                         memory_space=pltpu.SMEM),
            pl.BlockSpec((1, 1, ROW_BLOCK, 1), lambda k, rb: (k, rb, 0, 0)),
        ],
        out_specs=pl.BlockSpec((ROW_BLOCK, 1, 1, GROUP),
                               lambda k, rb: (rb, k, 0, 0)),
        out_shape=jax.ShapeDtypeStruct((B, NUM_CHANNELS, 1, GROUP),
                                       jnp.float32),
        scratch_shapes=[pltpu.VMEM((ROW_BLOCK, 1, GROUP), jnp.float32)],
        compiler_params=pltpu.CompilerParams(
            dimension_semantics=("parallel", "arbitrary")),
    )(z4, perm4, lam4)


def kernel(z_style, subject_labels):
    labels = subject_labels.astype(jnp.int32)
    key = jax.random.key(42)
    k_lam, k_g = jax.random.split(key)
    lam4 = jax.random.beta(
        k_lam, EPOCH_ALPHA, EPOCH_ALPHA,
        (NUM_CHANNELS, B, 1)).astype(jnp.float32).reshape(
            NUM_CHANNELS, N_ROW_BLOCKS, ROW_BLOCK, 1)
    keys = jnp.stack(
        [jax.random.key_data(jax.random.fold_in(k_g, k))
         for k in range(NUM_CHANNELS)])

    perm = _compute_perm(keys, labels)
    perm4 = perm.reshape(NUM_CHANNELS, N_ROW_BLOCKS, 1, ROW_BLOCK)
    z4 = z_style.reshape(B, NUM_CHANNELS, 1, GROUP)
    out4 = _mix(z4, perm4, lam4)
    return out4.reshape(B, STYLE_DIM)


# speedup vs baseline: 1.0131x; 1.0131x over previous
import jax
import jax.numpy as jnp
from jax.experimental import pallas as pl
from jax.experimental.pallas import tpu as pltpu

NUM_CHANNELS = 8
EPOCH_ALPHA = 0.3
B = 8192
STYLE_DIM = 4096
GROUP = STYLE_DIM // NUM_CHANNELS


ROWS_PER_STEP = 64
LANES = 128
N_ROW_STEPS = B // ROWS_PER_STEP
N_COL_ITERS = B // LANES

_ROT_A = (13, 15, 26, 6)
_ROT_B = (17, 29, 16, 24)


def _perm_kernel(keys_ref, cand_lab_ref, row_lab_ref, out_ref):
    k = pl.program_id(0)
    s = pl.program_id(1)
    row_base = s * ROWS_PER_STEP

    k1 = keys_ref[k, 0]
    k2 = keys_ref[k, 1]
    k3 = k1 ^ k2 ^ jnp.uint32(0x1BD11BDA)
    ks = (k1, k2, k3)

    row_lab = row_lab_ref[0]
    rows_i32 = (jax.lax.broadcasted_iota(jnp.int32, (ROWS_PER_STEP, 1), 0)
                + row_base)
    row_term = rows_i32.astype(jnp.uint32) * jnp.uint32(B)
    lane_u = jax.lax.broadcasted_iota(jnp.uint32, (1, LANES), 1)
    lane_i32 = jax.lax.broadcasted_iota(jnp.int32, (1, LANES), 1)
    x1_base = (row_term + lane_u) + ks[1]

    def one_chunk(jj):
        cand_lab = cand_lab_ref[jj, 0]
        x1 = x1_base + (jj * LANES).astype(jnp.uint32)
        x0 = jnp.full((ROWS_PER_STEP, LANES), ks[0], dtype=jnp.uint32)
        for i in range(5):
            rots = _ROT_A if i % 2 == 0 else _ROT_B
            for r in rots:
                x0 = x0 + x1
                x1 = ((x1 << jnp.uint32(r)) | (x1 >> jnp.uint32(32 - r)))
                x1 = x1 ^ x0
            x0 = x0 + ks[(i + 1) % 3]
            x1 = x1 + (ks[(i + 2) % 3] + jnp.uint32(i + 1))
        score = ((x0 ^ x1) >> jnp.uint32(9)).astype(jnp.int32)
        valid = cand_lab[None, :] != row_lab
        return jnp.where(valid, score, -1)

    UNROLL = 2

    def body(g, carry):
        run_max, run_idx = carry
        for u in range(UNROLL):
            jj = g * UNROLL + u
            masked = one_chunk(jj)
            take = masked > run_max
            run_max = jnp.maximum(run_max, masked)
            run_idx = jnp.where(take, lane_i32 + jj * LANES, run_idx)
        return (run_max, run_idx)

    init = (jnp.full((ROWS_PER_STEP, LANES), -1, jnp.int32),
            jnp.zeros((ROWS_PER_STEP, LANES), jnp.int32))
    run_max, run_idx = jax.lax.fori_loop(0, N_COL_ITERS // UNROLL, body, init)
    rmax = jnp.max(run_max, axis=1, keepdims=True)
    ridx = jnp.min(jnp.where(run_max == rmax, run_idx, jnp.int32(B)),
                   axis=1, keepdims=True)
    out_ref[0, 0] = jnp.where(rmax >= 0, ridx, rows_i32)


def _compute_perm(keys, labels):
    cand_lab = labels.reshape(N_COL_ITERS, 1, LANES)
    row_lab = labels.reshape(N_ROW_STEPS, ROWS_PER_STEP, 1)
    out = pl.pallas_call(
        _perm_kernel,
        grid=(NUM_CHANNELS, N_ROW_STEPS),
        in_specs=[
            pl.BlockSpec((NUM_CHANNELS, 2), lambda k, s: (0, 0),
                         memory_space=pltpu.SMEM),
            pl.BlockSpec((N_COL_ITERS, 1, LANES), lambda k, s: (0, 0, 0)),
            pl.BlockSpec((1, ROWS_PER_STEP, 1), lambda k, s: (s, 0, 0)),
        ],
        out_specs=pl.BlockSpec((1, 1, ROWS_PER_STEP, 1),
                               lambda k, s: (k, s, 0, 0)),
        out_shape=jax.ShapeDtypeStruct(
            (NUM_CHANNELS, N_ROW_STEPS, ROWS_PER_STEP, 1), jnp.int32),
        compiler_params=pltpu.CompilerParams(
            dimension_semantics=("parallel", "arbitrary")),
    )(keys, cand_lab, row_lab)
    return out.reshape(NUM_CHANNELS, B)


ROW_BLOCK = 1024
N_ROW_BLOCKS = B // ROW_BLOCK
GATHER_UNROLL = 8


def _mix_kernel(src_ref, perm_ref, lam_ref, out_ref):
    rb = pl.program_id(1)
    row_start = rb * ROW_BLOCK

    def body(g, _):
        for u in range(GATHER_UNROLL):
            mi = g * GATHER_UNROLL + u
            didx = perm_ref[0, 0, 0, mi]
            l = lam_ref[0, 0, 0, mi]
            self_row = src_ref[row_start + mi, 0, 0]
            donor = src_ref[didx, 0, 0]
            out_ref[mi, 0, 0] = l * self_row + (jnp.float32(1.0) - l) * donor
        return 0

    jax.lax.fori_loop(0, ROW_BLOCK // GATHER_UNROLL, body, 0)


def _mix(z4, perm, lam2):
    return pl.pallas_call(
        _mix_kernel,
        grid=(NUM_CHANNELS, N_ROW_BLOCKS),
        in_specs=[
            pl.BlockSpec((B, 1, 1, GROUP), lambda k, rb: (0, k, 0, 0)),
            pl.BlockSpec((1, 1, 1, ROW_BLOCK), lambda k, rb: (k, rb, 0, 0),
                         memory_space=pltpu.SMEM),
            pl.BlockSpec((1, 1, 1, ROW_BLOCK), lambda k, rb: (k, rb, 0, 0),
                         memory_space=pltpu.SMEM),
        ],
        out_specs=pl.BlockSpec((ROW_BLOCK, 1, 1, GROUP),
                               lambda k, rb: (rb, k, 0, 0)),
        out_shape=jax.ShapeDtypeStruct((B, NUM_CHANNELS, 1, GROUP),
                                       jnp.float32),
        compiler_params=pltpu.CompilerParams(
            dimension_semantics=("parallel", "arbitrary")),
    )(z4, perm.reshape(NUM_CHANNELS, N_ROW_BLOCKS, 1, ROW_BLOCK),
      lam2.reshape(NUM_CHANNELS, N_ROW_BLOCKS, 1, ROW_BLOCK))


def kernel(z_style, subject_labels):
    labels = subject_labels.astype(jnp.int32)
    key = jax.random.key(42)
    k_lam, k_g = jax.random.split(key)
    lam2 = jax.random.beta(
        k_lam, EPOCH_ALPHA, EPOCH_ALPHA,
        (NUM_CHANNELS, B, 1)).astype(jnp.float32)[..., 0]
    keys = jnp.stack(
        [jax.random.key_data(jax.random.fold_in(k_g, k))
         for k in range(NUM_CHANNELS)])

    perm = _compute_perm(keys, labels)
    z4 = z_style.reshape(B, NUM_CHANNELS, 1, GROUP)
    out4 = _mix(z4, perm, lam2)
    return out4.reshape(B, STYLE_DIM)


# speedup vs baseline: 1.0375x; 1.0240x over previous
import jax
import jax.numpy as jnp
from jax.experimental import pallas as pl
from jax.experimental.pallas import tpu as pltpu

NUM_CHANNELS = 8
EPOCH_ALPHA = 0.3
B = 8192
STYLE_DIM = 4096
GROUP = STYLE_DIM // NUM_CHANNELS


ROWS_PER_STEP = 64
LANES = 128
N_ROW_STEPS = B // ROWS_PER_STEP
N_COL_ITERS = B // LANES

_ROT_A = (13, 15, 26, 6)
_ROT_B = (17, 29, 16, 24)


def _perm_kernel(keys_ref, cand_lab_ref, row_lab_ref, rm_ref, ri_ref):
    k = pl.program_id(0)
    s = pl.program_id(1)
    row_base = s * ROWS_PER_STEP

    k1 = keys_ref[k, 0]
    k2 = keys_ref[k, 1]
    k3 = k1 ^ k2 ^ jnp.uint32(0x1BD11BDA)
    ks = (k1, k2, k3)

    row_lab = row_lab_ref[0]
    rows_i32 = (jax.lax.broadcasted_iota(jnp.int32, (ROWS_PER_STEP, 1), 0)
                + row_base)
    row_term = rows_i32.astype(jnp.uint32) * jnp.uint32(B)
    lane_u = jax.lax.broadcasted_iota(jnp.uint32, (1, LANES), 1)
    lane_i32 = jax.lax.broadcasted_iota(jnp.int32, (1, LANES), 1)
    x1_base = (row_term + lane_u) + ks[1]

    def one_chunk(jj):
        cand_lab = cand_lab_ref[jj, 0]
        x1 = x1_base + (jj * LANES).astype(jnp.uint32)
        x0 = jnp.full((ROWS_PER_STEP, LANES), ks[0], dtype=jnp.uint32)
        for i in range(5):
            rots = _ROT_A if i % 2 == 0 else _ROT_B
            for r in rots:
                x0 = x0 + x1
                x1 = ((x1 << jnp.uint32(r)) | (x1 >> jnp.uint32(32 - r)))
                x1 = x1 ^ x0
            x0 = x0 + ks[(i + 1) % 3]
            x1 = x1 + (ks[(i + 2) % 3] + jnp.uint32(i + 1))
        score = ((x0 ^ x1) >> jnp.uint32(9)).astype(jnp.int32)
        valid = cand_lab[None, :] != row_lab
        return jnp.where(valid, score, -1)

    UNROLL = 2

    def body(g, carry):
        run_max, run_idx = carry
        for u in range(UNROLL):
            jj = g * UNROLL + u
            masked = one_chunk(jj)
            take = masked > run_max
            run_max = jnp.maximum(run_max, masked)
            run_idx = jnp.where(take, lane_i32 + jj * LANES, run_idx)
        return (run_max, run_idx)

    init = (jnp.full((ROWS_PER_STEP, LANES), -1, jnp.int32),
            jnp.zeros((ROWS_PER_STEP, LANES), jnp.int32))
    run_max, run_idx = jax.lax.fori_loop(0, N_COL_ITERS // UNROLL, body, init)
    rm_ref[0, 0] = run_max
    ri_ref[0, 0] = run_idx


FIN_GROUPS = 16


def _finalize_kernel(rm_ref, ri_ref, out_ref):
    s = pl.program_id(1)
    for rg in range(FIN_GROUPS):
        run_max = rm_ref[0, rg]
        run_idx = ri_ref[0, rg]
        rmax = jnp.max(run_max, axis=1, keepdims=True)
        ridx = jnp.min(jnp.where(run_max == rmax, run_idx, jnp.int32(B)),
                       axis=1, keepdims=True)
        rows_i32 = (jax.lax.broadcasted_iota(jnp.int32, (ROWS_PER_STEP, 1), 0)
                    + (s * FIN_GROUPS + rg) * ROWS_PER_STEP)
        out_ref[0, rg] = jnp.where(rmax >= 0, ridx, rows_i32)


def _compute_perm(keys, labels):
    cand_lab = labels.reshape(N_COL_ITERS, 1, LANES)
    row_lab = labels.reshape(N_ROW_STEPS, ROWS_PER_STEP, 1)
    rm, ri = pl.pallas_call(
        _perm_kernel,
        grid=(NUM_CHANNELS, N_ROW_STEPS),
        in_specs=[
            pl.BlockSpec((NUM_CHANNELS, 2), lambda k, s: (0, 0),
                         memory_space=pltpu.SMEM),
            pl.BlockSpec((N_COL_ITERS, 1, LANES), lambda k, s: (0, 0, 0)),
            pl.BlockSpec((1, ROWS_PER_STEP, 1), lambda k, s: (s, 0, 0)),
        ],
        out_specs=[
            pl.BlockSpec((1, 1, ROWS_PER_STEP, LANES),
                         lambda k, s: (k, s, 0, 0)),
            pl.BlockSpec((1, 1, ROWS_PER_STEP, LANES),
                         lambda k, s: (k, s, 0, 0)),
        ],
        out_shape=[
            jax.ShapeDtypeStruct(
                (NUM_CHANNELS, N_ROW_STEPS, ROWS_PER_STEP, LANES), jnp.int32),
            jax.ShapeDtypeStruct(
                (NUM_CHANNELS, N_ROW_STEPS, ROWS_PER_STEP, LANES), jnp.int32),
        ],
        compiler_params=pltpu.CompilerParams(
            dimension_semantics=("parallel", "arbitrary")),
    )(keys, cand_lab, row_lab)
    out = pl.pallas_call(
        _finalize_kernel,
        grid=(NUM_CHANNELS, N_ROW_STEPS // FIN_GROUPS),
        in_specs=[
            pl.BlockSpec((1, FIN_GROUPS, ROWS_PER_STEP, LANES),
                         lambda k, s: (k, s, 0, 0)),
            pl.BlockSpec((1, FIN_GROUPS, ROWS_PER_STEP, LANES),
                         lambda k, s: (k, s, 0, 0)),
        ],
        out_specs=pl.BlockSpec((1, FIN_GROUPS, ROWS_PER_STEP, 1),
                               lambda k, s: (k, s, 0, 0)),
        out_shape=jax.ShapeDtypeStruct(
            (NUM_CHANNELS, N_ROW_STEPS, ROWS_PER_STEP, 1), jnp.int32),
        compiler_params=pltpu.CompilerParams(
            dimension_semantics=("parallel", "arbitrary")),
    )(rm, ri)
    return out.reshape(NUM_CHANNELS, B)


ROW_BLOCK = 1024
N_ROW_BLOCKS = B // ROW_BLOCK
GATHER_UNROLL = 8


def _mix_kernel(src_ref, perm_ref, lam_ref, out_ref):
    rb = pl.program_id(1)
    row_start = rb * ROW_BLOCK

    def body(g, _):
        for u in range(GATHER_UNROLL):
            mi = g * GATHER_UNROLL + u
            didx = perm_ref[0, 0, 0, mi]
            l = lam_ref[0, 0, 0, mi]
            self_row = src_ref[row_start + mi, 0, 0]
            donor = src_ref[didx, 0, 0]
            out_ref[mi, 0, 0] = l * self_row + (jnp.float32(1.0) - l) * donor
        return 0

    jax.lax.fori_loop(0, ROW_BLOCK // GATHER_UNROLL, body, 0)


def _mix(z4, perm, lam2):
    return pl.pallas_call(
        _mix_kernel,
        grid=(NUM_CHANNELS, N_ROW_BLOCKS),
        in_specs=[
            pl.BlockSpec((B, 1, 1, GROUP), lambda k, rb: (0, k, 0, 0)),
            pl.BlockSpec((1, 1, 1, ROW_BLOCK), lambda k, rb: (k, rb, 0, 0),
                         memory_space=pltpu.SMEM),
            pl.BlockSpec((1, 1, 1, ROW_BLOCK), lambda k, rb: (k, rb, 0, 0),
                         memory_space=pltpu.SMEM),
        ],
        out_specs=pl.BlockSpec((ROW_BLOCK, 1, 1, GROUP),
                               lambda k, rb: (rb, k, 0, 0)),
        out_shape=jax.ShapeDtypeStruct((B, NUM_CHANNELS, 1, GROUP),
                                       jnp.float32),
        compiler_params=pltpu.CompilerParams(
            dimension_semantics=("parallel", "arbitrary")),
    )(z4, perm.reshape(NUM_CHANNELS, N_ROW_BLOCKS, 1, ROW_BLOCK),
      lam2.reshape(NUM_CHANNELS, N_ROW_BLOCKS, 1, ROW_BLOCK))


def kernel(z_style, subject_labels):
    labels = subject_labels.astype(jnp.int32)
    key = jax.random.key(42)
    k_lam, k_g = jax.random.split(key)
    lam2 = jax.random.beta(
        k_lam, EPOCH_ALPHA, EPOCH_ALPHA,
        (NUM_CHANNELS, B, 1)).astype(jnp.float32)[..., 0]
    keys = jnp.stack(
        [jax.random.key_data(jax.random.fold_in(k_g, k))
         for k in range(NUM_CHANNELS)])

    perm = _compute_perm(keys, labels)
    z4 = z_style.reshape(B, NUM_CHANNELS, 1, GROUP)
    out4 = _mix(z4, perm, lam2)
    return out4.reshape(B, STYLE_DIM)
